# Initial kernel scaffold; baseline (speedup 1.0000x reference)
import jax
import jax.numpy as jnp
from jax.experimental import pallas as pl


def kernel(x, triples, W_mlp, b_mlp, basis0, att0, bias0, basis1, att1, bias1, rel_emb):
    raise NotImplementedError("write your pallas kernel here")


# speedup vs baseline: 7.4128x; 7.4128x over previous
import jax
import jax.numpy as jnp
from jax import lax
from jax.experimental import pallas as pl
from jax.experimental.pallas import tpu as pltpu
from jax.experimental.pallas import tpu_sc as plsc

N = 10000
R = 16
B = 4
D = 128
E = 320000

NC = 2
NS = 16
NW = NC * NS
K = 160
NCHUNK = 63
EPW = K * NCHUNK
EP = EPW * NW
NPAD = 10240
NPT = NPAD // NS


def _wcat_body(att0_ref, basis0_ref, att1_ref, basis1_ref, w0_ref, w1_ref):
    for r in range(R):
        acc0 = att0_ref[r, 0] * basis0_ref[0]
        acc1 = att1_ref[r, 0] * basis1_ref[0]
        for b in range(1, B):
            acc0 = acc0 + att0_ref[r, b] * basis0_ref[b]
            acc1 = acc1 + att1_ref[r, b] * basis1_ref[b]
        w0_ref[:, r * D:(r + 1) * D] = acc0
        w1_ref[:, r * D:(r + 1) * D] = acc1


def _build_wcat(att0, basis0, att1, basis1):
    return pl.pallas_call(
        _wcat_body,
        out_shape=[jax.ShapeDtypeStruct((D, R * D), jnp.float32)] * 2,
    )(att0, basis0, att1, basis1)


BN = 1000


def _mlp_matmul_body(x_ref, wm_ref, bm_ref, wc_ref, out_ref):
    h = lax.dot_general(x_ref[...], wm_ref[...], (((1,), (1,)), ((), ())),
                        preferred_element_type=jnp.float32)
    h = h + bm_ref[...]
    out_ref[...] = jnp.dot(h, wc_ref[...], preferred_element_type=jnp.float32)


def _mlp_matmul(x, W_mlp, b_mlp, Wcat):
    return pl.pallas_call(
        _mlp_matmul_body,
        grid=(N // BN,),
        in_specs=[
            pl.BlockSpec((BN, D), lambda i: (i, 0)),
            pl.BlockSpec((D, D), lambda i: (0, 0)),
            pl.BlockSpec((1, D), lambda i: (0, 0)),
            pl.BlockSpec((D, R * D), lambda i: (0, 0)),
        ],
        out_specs=pl.BlockSpec((BN, R * D), lambda i: (i, 0)),
        out_shape=jax.ShapeDtypeStruct((N, R * D), jnp.float32),
    )(x, W_mlp, b_mlp.reshape(1, D), Wcat)


def _combine_matmul_body(part_ref, bias_ref, wc_ref, out_ref):
    h = part_ref[0] + part_ref[1] + bias_ref[...]
    h = jnp.maximum(h, 0.0)
    out_ref[...] = jnp.dot(h, wc_ref[...], preferred_element_type=jnp.float32)


def _combine_matmul(part, bias, Wcat):
    return pl.pallas_call(
        _combine_matmul_body,
        grid=(N // BN,),
        in_specs=[
            pl.BlockSpec((2, BN, D), lambda i: (0, i, 0)),
            pl.BlockSpec((1, D), lambda i: (0, 0)),
            pl.BlockSpec((D, R * D), lambda i: (0, 0)),
        ],
        out_specs=pl.BlockSpec((BN, R * D), lambda i: (i, 0)),
        out_shape=jax.ShapeDtypeStruct((N, R * D), jnp.float32),
    )(part, bias.reshape(1, D), Wcat)


def _final_body(part_ref, bias_ref, out_ref):
    out_ref[...] = part_ref[0] + part_ref[1] + bias_ref[...]


def _final(part, bias):
    return pl.pallas_call(
        _final_body,
        grid=(N // BN,),
        in_specs=[
            pl.BlockSpec((2, BN, D), lambda i: (0, i, 0)),
            pl.BlockSpec((1, D), lambda i: (0, 0)),
        ],
        out_specs=pl.BlockSpec((BN, D), lambda i: (i, 0)),
        out_shape=jax.ShapeDtypeStruct((N, D), jnp.float32),
    )(part, bias.reshape(1, D))


def _dinv_body(degp_ref, out_ref):
    out_ref[...] = 1.0 / (degp_ref[0] + degp_ref[1])


def _merge_dinv(degp):
    return pl.pallas_call(
        _dinv_body,
        grid=(NPAD // 1024,),
        in_specs=[pl.BlockSpec((2, 1024, D), lambda i: (0, i, 0))],
        out_specs=pl.BlockSpec((1024, D), lambda i: (i, 0)),
        out_shape=jax.ShapeDtypeStruct((NPAD, D), jnp.float32),
    )(degp)


def _iota16():
    return lax.iota(jnp.int32, 16)


def _splat(vec, j):
    return vec.at[jnp.full((16,), j, jnp.int32)].get(mode="promise_in_bounds")


def _sc_deg_body(sidx_hbm, rel_hbm, zer_hbm, out_hbm, dacc, sv, rv, oh):
    cid = lax.axis_index("c")
    sid = lax.axis_index("s")
    wid = cid * NS + sid
    pltpu.sync_copy(zer_hbm, dacc.at[pl.ds(sid * NPT, NPT)])
    pltpu.sync_copy(zer_hbm.at[pl.ds(0, K)], oh)
    plsc.subcore_barrier()
    base = wid * EPW

    def chunk(c, _):
        off = pl.multiple_of(base + c * K, 8)
        pltpu.sync_copy(sidx_hbm.at[pl.ds(off, K)], sv)
        pltpu.sync_copy(rel_hbm.at[pl.ds(off, K)], rv)

        def grp(g, _):
            rel16 = rv[pl.ds(g * 16, 16)]
            for j in range(16):
                rs = _splat(rel16, j)
                row = jnp.where(_iota16() == rs, 1.0, 0.0)
                oh[g * 16 + j, pl.ds(0, 16)] = row
            return 0

        lax.fori_loop(0, K // 16, grp, 0)
        pltpu.sync_copy(oh, dacc.at[sv], add=True)
        return 0

    lax.fori_loop(0, NCHUNK, chunk, 0)
    plsc.subcore_barrier()
    pltpu.sync_copy(dacc.at[pl.ds(sid * NPT, NPT)],
                    out_hbm.at[cid, pl.ds(sid * NPT, NPT)])


def _sc_deg(sidx, rel, zer128):
    mesh = plsc.VectorSubcoreMesh(core_axis_name="c", subcore_axis_name="s")
    return pl.kernel(
        _sc_deg_body,
        out_type=jax.ShapeDtypeStruct((NC, NPAD, D), jnp.float32),
        mesh=mesh,
        scratch_types=[
            pltpu.VMEM_SHARED((NPAD, D), jnp.float32),
            pltpu.VMEM((K,), jnp.int32),
            pltpu.VMEM((K,), jnp.int32),
            pltpu.VMEM((K, D), jnp.float32),
        ],
    )(sidx, rel, zer128)


def _sc_agg0_body(hcat_hbm, gidx_hbm, sidx_hbm, rel_hbm, dinv_hbm, zer_hbm,
                  out_hbm, val_hbm, acc, gv, sv, rv, vv, rows, drows,
                  gsem, dsem):
    cid = lax.axis_index("c")
    sid = lax.axis_index("s")
    wid = cid * NS + sid
    pltpu.sync_copy(zer_hbm, acc.at[pl.ds(sid * NPT, NPT)])
    plsc.subcore_barrier()
    base = wid * EPW

    def chunk(c, _):
        off = pl.multiple_of(base + c * K, 8)
        pltpu.sync_copy(gidx_hbm.at[pl.ds(off, K)], gv)
        pltpu.sync_copy(sidx_hbm.at[pl.ds(off, K)], sv)
        pltpu.sync_copy(rel_hbm.at[pl.ds(off, K)], rv)
        cg = pltpu.async_copy(hcat_hbm.at[gv], rows, gsem)
        cd = pltpu.async_copy(dinv_hbm.at[sv], drows, dsem)
        cg.wait()
        cd.wait()

        def grp(g, _):
            rel16 = rv[pl.ds(g * 16, 16)]
            vacc = jnp.zeros((16,), jnp.float32)
            for j in range(16):
                e = g * 16 + j
                rs = _splat(rel16, j)
                drow = drows[e, pl.ds(0, 16)]
                s = drow.at[rs].get(mode="promise_in_bounds")
                vacc = jnp.where(_iota16() == j, s, vacc)
                for q in range(D // 16):
                    sl = pl.ds(q * 16, 16)
                    rows[e, sl] = rows[e, sl] * s
            vv[pl.ds(g * 16, 16)] = vacc
            return 0

        lax.fori_loop(0, K // 16, grp, 0)
        pltpu.sync_copy(vv, val_hbm.at[pl.ds(off, K)])
        pltpu.sync_copy(rows, acc.at[sv], add=True)
        return 0

    lax.fori_loop(0, NCHUNK, chunk, 0)
    plsc.subcore_barrier()
    pltpu.sync_copy(acc.at[pl.ds(sid * NPT, NPT)],
                    out_hbm.at[cid, pl.ds(sid * NPT, NPT)])


def _sc_agg0(hcat_rows, gidx, sidx, rel, dinv, zer128):
    mesh = plsc.VectorSubcoreMesh(core_axis_name="c", subcore_axis_name="s")
    return pl.kernel(
        _sc_agg0_body,
        out_type=[jax.ShapeDtypeStruct((NC, NPAD, D), jnp.float32),
                  jax.ShapeDtypeStruct((EP,), jnp.float32)],
        mesh=mesh,
        scratch_types=[
            pltpu.VMEM_SHARED((NPAD, D), jnp.float32),
            pltpu.VMEM((K,), jnp.int32),
            pltpu.VMEM((K,), jnp.int32),
            pltpu.VMEM((K,), jnp.int32),
            pltpu.VMEM((K,), jnp.float32),
            pltpu.VMEM((K, D), jnp.float32),
            pltpu.VMEM((K, D), jnp.float32),
            pltpu.SemaphoreType.DMA,
            pltpu.SemaphoreType.DMA,
        ],
    )(hcat_rows, gidx, sidx, rel, dinv, zer128)


def _sc_agg1_body(hcat_hbm, gidx_hbm, sidx_hbm, val_hbm, zer_hbm,
                  out_hbm, acc, gv, sv, vv, rows, gsem):
    cid = lax.axis_index("c")
    sid = lax.axis_index("s")
    wid = cid * NS + sid
    pltpu.sync_copy(zer_hbm, acc.at[pl.ds(sid * NPT, NPT)])
    plsc.subcore_barrier()
    base = wid * EPW

    def chunk(c, _):
        off = pl.multiple_of(base + c * K, 8)
        pltpu.sync_copy(gidx_hbm.at[pl.ds(off, K)], gv)
        pltpu.sync_copy(sidx_hbm.at[pl.ds(off, K)], sv)
        pltpu.sync_copy(val_hbm.at[pl.ds(off, K)], vv)
        pltpu.async_copy(hcat_hbm.at[gv], rows, gsem).wait()

        def grp(g, _):
            val16 = vv[pl.ds(g * 16, 16)]
            for j in range(16):
                e = g * 16 + j
                s = _splat(val16, j)
                for q in range(D // 16):
                    sl = pl.ds(q * 16, 16)
                    rows[e, sl] = rows[e, sl] * s
            return 0

        lax.fori_loop(0, K // 16, grp, 0)
        pltpu.sync_copy(rows, acc.at[sv], add=True)
        return 0

    lax.fori_loop(0, NCHUNK, chunk, 0)
    plsc.subcore_barrier()
    pltpu.sync_copy(acc.at[pl.ds(sid * NPT, NPT)],
                    out_hbm.at[cid, pl.ds(sid * NPT, NPT)])


def _sc_agg1(hcat_rows, gidx, sidx, val, zer128):
    mesh = plsc.VectorSubcoreMesh(core_axis_name="c", subcore_axis_name="s")
    return pl.kernel(
        _sc_agg1_body,
        out_type=jax.ShapeDtypeStruct((NC, NPAD, D), jnp.float32),
        mesh=mesh,
        scratch_types=[
            pltpu.VMEM_SHARED((NPAD, D), jnp.float32),
            pltpu.VMEM((K,), jnp.int32),
            pltpu.VMEM((K,), jnp.int32),
            pltpu.VMEM((K,), jnp.float32),
            pltpu.VMEM((K, D), jnp.float32),
            pltpu.SemaphoreType.DMA,
        ],
    )(hcat_rows, gidx, sidx, val, zer128)


def kernel(x, triples, W_mlp, b_mlp, basis0, att0, bias0, basis1, att1, bias1,
           rel_emb):
    src = triples[:, 0]
    rel = triples[:, 1]
    dst = triples[:, 2]
    npad = EP - E
    src = jnp.concatenate([src, jnp.full((npad,), NPAD - 1, jnp.int32)])
    rel = jnp.concatenate([rel, jnp.zeros((npad,), jnp.int32)])
    dst = jnp.concatenate([dst, jnp.zeros((npad,), jnp.int32)])
    gidx = dst * R + rel

    zer128 = jnp.zeros((NPT, D), jnp.float32)

    Wcat0, Wcat1 = _build_wcat(att0, basis0, att1, basis1)

    degp = _sc_deg(src, rel, zer128)
    dinv = _merge_dinv(degp)

    hcat0 = _mlp_matmul(x, W_mlp, b_mlp, Wcat0)
    part0, val = _sc_agg0(hcat0.reshape(N * R, D), gidx, src, rel, dinv,
                          zer128)

    hcat1 = _combine_matmul(part0, bias0, Wcat1)
    part1 = _sc_agg1(hcat1.reshape(N * R, D), gidx, src, val, zer128)

    out = _final(part1, bias1)
    return (out, rel_emb)
